# Initial kernel scaffold; baseline (speedup 1.0000x reference)
import jax
import jax.numpy as jnp
from jax.experimental import pallas as pl


def kernel(blob, edge_index, W_ih, W_hh, b_ih, b_hh, W0, b0, W1, b1, W2, b2):
    raise NotImplementedError("write your pallas kernel here")


# speedup vs baseline: 1.0171x; 1.0171x over previous
import functools

import jax
import jax.numpy as jnp
from jax import lax
from jax.experimental import pallas as pl
from jax.experimental.pallas import tpu as pltpu
from jax.experimental.pallas import tpu_sc as plsc

N = 10000
E = 160000
H = 256
T = 12
NPAD = 10240
NW = 32
NB = NPAD // NW
NC = 2

_mesh = plsc.VectorSubcoreMesh(core_axis_name="c", subcore_axis_name="s")


def _scalar_at(ref, i):
    v = plsc.load_gather(ref, [jnp.full((16,), i, jnp.int32)])
    return jnp.max(v)


@functools.partial(
    pl.kernel,
    out_type=jax.ShapeDtypeStruct((2 * NPAD,), jnp.float32),
    mesh=_mesh,
    compiler_params=pltpu.CompilerParams(needs_layout_passes=False),
    scratch_types=[
        pltpu.VMEM((16 * (NB + 1),), jnp.float32),
        pltpu.VMEM((NB,), jnp.float32),
        pltpu.VMEM((40,), jnp.int32),
        pltpu.VMEM((16,), jnp.int32),
    ],
)
def _sc_degrees(dst_s, src_ss, bounds_d, bounds_s, out, hist, degbuf, bvm, ivm):
    wid = lax.axis_index("s") * NC + lax.axis_index("c")
    iota = lax.iota(jnp.int32, 16)
    ones = jnp.full((16,), 1.0, jnp.float32)
    zeros16 = jnp.zeros((16,), jnp.float32)
    for p, (arr, bnds) in enumerate(((dst_s, bounds_d), (src_ss, bounds_s))):
        pltpu.sync_copy(bnds, bvm)
        s0 = _scalar_at(bvm, wid)
        s1 = _scalar_at(bvm, wid + 1)

        def zbody(i, _):
            hist[pl.ds(i * 16, 16)] = zeros16
            return 0
        lax.fori_loop(0, NB + 1, zbody, 0)

        base_node = wid * NB

        def cbody(c):
            cc = pl.multiple_of(c, 16)
            pltpu.sync_copy(arr.at[pl.ds(cc, 16)], ivm)
            d = ivm[...]
            pos = c + iota
            valid = (pos >= s0) & (pos < s1)
            local = jnp.where(valid, d - base_node, NB)
            idx = iota * (NB + 1) + local
            plsc.addupdate_scatter(hist, [idx], ones)
            return c + 16

        c0 = (s0 // 16) * 16
        lax.while_loop(lambda c: c < s1, cbody, c0)

        for g in range(NB // 16):
            acc = zeros16
            for lane in range(16):
                acc = acc + hist[pl.ds(lane * (NB + 1) + g * 16, 16)]
            degbuf[pl.ds(g * 16, 16)] = acc
        pltpu.sync_copy(
            degbuf,
            out.at[pl.ds(pl.multiple_of(p * NPAD + wid * NB, NB), NB)])


@functools.partial(
    pl.kernel,
    out_type=jax.ShapeDtypeStruct((T * NPAD * H,), jnp.float32),
    mesh=_mesh,
    compiler_params=pltpu.CompilerParams(needs_layout_passes=False),
    scratch_types=[
        pltpu.VMEM(((NB + 1) * H,), jnp.float32),
        pltpu.VMEM((16, H), jnp.float32),
        pltpu.VMEM((40,), jnp.int32),
        pltpu.VMEM((16,), jnp.int32),
        pltpu.VMEM((16,), jnp.int32),
        pltpu.VMEM((16,), jnp.int32),
        pltpu.SemaphoreType.DMA,
    ],
)
def _sc_agg256(table, src_s, dst_s, bounds, out,
               acc, stage, bvm, svm, dvm, gidx, sem):
    wid = lax.axis_index("s") * NC + lax.axis_index("c")
    iota = lax.iota(jnp.int32, 16)
    zeros16 = jnp.zeros((16,), jnp.float32)
    pltpu.sync_copy(bounds, bvm)
    s0 = _scalar_at(bvm, wid)
    s1 = _scalar_at(bvm, wid + 1)
    base_node = wid * NB
    c0 = (s0 // 16) * 16

    for t in range(T):
        def zbody(i, _):
            acc[pl.ds(i * 16, 16)] = zeros16
            return 0
        lax.fori_loop(0, (NB + 1) * H // 16, zbody, 0)

        def cbody(c):
            cc = pl.multiple_of(c, 16)
            pltpu.sync_copy(src_s.at[pl.ds(cc, 16)], svm)
            pltpu.sync_copy(dst_s.at[pl.ds(cc, 16)], dvm)
            pos = c + iota
            valid = (pos >= s0) & (pos < s1)
            local = jnp.where(valid, dvm[...] - base_node, NB)
            gidx[...] = svm[...] + (t * NPAD)
            pltpu.async_copy(table.at[gidx], stage, sem).wait()
            for j in range(16):
                lj = jnp.max(jnp.where(iota == j, local, 0))
                rowbase = jnp.full((16,), lj * H, jnp.int32)
                for k in range(H // 16):
                    idxv = rowbase + (iota + k * 16)
                    plsc.addupdate_scatter(acc, [idxv],
                                           stage[j, pl.ds(k * 16, 16)])
            return c + 16

        lax.while_loop(lambda c: c < s1, cbody, c0)
        pltpu.sync_copy(
            acc.at[pl.ds(0, NB * H)],
            out.at[pl.ds(pl.multiple_of((t * NPAD + wid * NB) * H, NB * H),
                         NB * H)])


@functools.partial(
    pl.kernel,
    out_type=jax.ShapeDtypeStruct((NPAD * 128,), jnp.float32),
    mesh=_mesh,
    compiler_params=pltpu.CompilerParams(needs_layout_passes=False),
    scratch_types=[
        pltpu.VMEM(((NB + 1) * 128,), jnp.float32),
        pltpu.VMEM((16, 128), jnp.float32),
        pltpu.VMEM((40,), jnp.int32),
        pltpu.VMEM((16,), jnp.int32),
        pltpu.VMEM((16,), jnp.int32),
        pltpu.SemaphoreType.DMA,
    ],
)
def _sc_agg16(table, src_s, dst_s, bounds, out,
              acc, stage, bvm, svm, dvm, sem):
    wid = lax.axis_index("s") * NC + lax.axis_index("c")
    iota = lax.iota(jnp.int32, 16)
    zeros16 = jnp.zeros((16,), jnp.float32)
    pltpu.sync_copy(bounds, bvm)
    s0 = _scalar_at(bvm, wid)
    s1 = _scalar_at(bvm, wid + 1)
    base_node = wid * NB

    def zbody(i, _):
        acc[pl.ds(i * 16, 16)] = zeros16
        return 0
    lax.fori_loop(0, (NB + 1) * 128 // 16, zbody, 0)

    def cbody(c):
        cc = pl.multiple_of(c, 16)
        pltpu.sync_copy(src_s.at[pl.ds(cc, 16)], svm)
        pltpu.sync_copy(dst_s.at[pl.ds(cc, 16)], dvm)
        pos = c + iota
        valid = (pos >= s0) & (pos < s1)
        local = jnp.where(valid, dvm[...] - base_node, NB)
        pltpu.async_copy(table.at[svm], stage, sem).wait()
        for j in range(16):
            lj = jnp.max(jnp.where(iota == j, local, 0))
            rowbase = jnp.full((16,), lj * 128, jnp.int32)
            for k in range(8):
                idxv = rowbase + (iota + k * 16)
                plsc.addupdate_scatter(acc, [idxv],
                                       stage[j, pl.ds(k * 16, 16)])
        return c + 16

    c0 = (s0 // 16) * 16
    lax.while_loop(lambda c: c < s1, cbody, c0)
    pltpu.sync_copy(acc.at[pl.ds(0, NB * 128)],
                    out.at[pl.ds(pl.multiple_of(wid * NB * 128, NB * 128),
                                 NB * 128)])


_BN = 512
_NBLK = NPAD // _BN


def _lstm_body(x_ref, wih_ref, whh_ref, b_ref, deg_ref, z_ref):
    bn = x_ref.shape[0]
    h = jnp.zeros((bn, H), jnp.float32)
    c = jnp.zeros((bn, H), jnp.float32)
    nsrc = lax.rsqrt(jnp.maximum(deg_ref[...], 1.0))
    x = x_ref[...]
    wih = wih_ref[...]
    b = b_ref[...]
    whh = whh_ref[...]
    for t in range(T):
        gates = (x[:, t:t + 1] * wih
                 + jnp.dot(h, whh, preferred_element_type=jnp.float32,
                 precision=lax.Precision.HIGHEST) + b)
        i_g = jax.nn.sigmoid(gates[:, 0 * H:1 * H])
        f_g = jax.nn.sigmoid(gates[:, 1 * H:2 * H])
        g_g = jnp.tanh(gates[:, 2 * H:3 * H])
        o_g = jax.nn.sigmoid(gates[:, 3 * H:4 * H])
        c = f_g * c + i_g * g_g
        h = o_g * jnp.tanh(c)
        z_ref[t] = nsrc * h


def _lstm(blob2, wih, whhT, bsum, outdeg):
    return pl.pallas_call(
        _lstm_body,
        grid=(_NBLK,),
        in_specs=[
            pl.BlockSpec((_BN, T), lambda b: (b, 0)),
            pl.BlockSpec((1, 4 * H), lambda b: (0, 0)),
            pl.BlockSpec((H, 4 * H), lambda b: (0, 0)),
            pl.BlockSpec((1, 4 * H), lambda b: (0, 0)),
            pl.BlockSpec((_BN, 1), lambda b: (b, 0)),
        ],
        out_specs=pl.BlockSpec((T, _BN, H), lambda b: (0, b, 0)),
        out_shape=jax.ShapeDtypeStruct((T, NPAD, H), jnp.float32),
    )(blob2, wih, whhT, bsum, outdeg)


def _conv_body(a_ref, ideg_ref, odeg_ref, w_ref, b_ref, z_ref):
    ndst = lax.rsqrt(jnp.maximum(ideg_ref[...], 1.0))
    nsrc = lax.rsqrt(jnp.maximum(odeg_ref[...], 1.0))
    a = a_ref[0]
    hh = jax.nn.relu(ndst * jnp.dot(a, w_ref[...],
                                    preferred_element_type=jnp.float32,
                 precision=lax.Precision.HIGHEST)
                     + b_ref[...])
    z_ref[0] = nsrc * hh


def _conv(a3, indeg, outdeg, w, brow):
    return pl.pallas_call(
        _conv_body,
        grid=(T, _NBLK),
        in_specs=[
            pl.BlockSpec((1, _BN, H), lambda t, b: (t, b, 0)),
            pl.BlockSpec((_BN, 1), lambda t, b: (b, 0)),
            pl.BlockSpec((_BN, 1), lambda t, b: (b, 0)),
            pl.BlockSpec((H, H), lambda t, b: (0, 0)),
            pl.BlockSpec((1, H), lambda t, b: (0, 0)),
        ],
        out_specs=pl.BlockSpec((1, _BN, H), lambda t, b: (t, b, 0)),
        out_shape=jax.ShapeDtypeStruct((T, NPAD, H), jnp.float32),
    )(a3, indeg, outdeg, w, brow)


def _conv2_body(a_ref, ideg_ref, odeg_ref, w1_ref, b1_ref, w2s_ref, z_ref):
    ndst = lax.rsqrt(jnp.maximum(ideg_ref[...], 1.0))
    nsrc = lax.rsqrt(jnp.maximum(odeg_ref[...], 1.0))
    acc = jnp.zeros((a_ref.shape[1], 128), jnp.float32)
    for t in range(T):
        h2 = jax.nn.relu(ndst * jnp.dot(a_ref[t], w1_ref[...],
                                        preferred_element_type=jnp.float32,
                 precision=lax.Precision.HIGHEST)
                         + b1_ref[...])
        acc = acc + jnp.dot(h2, w2s_ref[t], preferred_element_type=jnp.float32,
                 precision=lax.Precision.HIGHEST)
    z_ref[...] = nsrc * acc


def _conv2(a3, indeg, outdeg, w1, b1row, w2s):
    return pl.pallas_call(
        _conv2_body,
        grid=(_NBLK,),
        in_specs=[
            pl.BlockSpec((T, _BN, H), lambda b: (0, b, 0)),
            pl.BlockSpec((_BN, 1), lambda b: (b, 0)),
            pl.BlockSpec((_BN, 1), lambda b: (b, 0)),
            pl.BlockSpec((H, H), lambda b: (0, 0)),
            pl.BlockSpec((1, H), lambda b: (0, 0)),
            pl.BlockSpec((T, H, 128), lambda b: (0, 0, 0)),
        ],
        out_specs=pl.BlockSpec((_BN, 128), lambda b: (b, 0)),
        out_shape=jax.ShapeDtypeStruct((NPAD, 128), jnp.float32),
    )(a3, indeg, outdeg, w1, b1row, w2s)


def _final_body(a_ref, ideg_ref, b2_ref, o_ref):
    ndst = lax.rsqrt(jnp.maximum(ideg_ref[...], 1.0))
    o_ref[...] = ndst * a_ref[...] + b2_ref[...]


def _final(a2, indeg, b2row):
    return pl.pallas_call(
        _final_body,
        grid=(_NBLK,),
        in_specs=[
            pl.BlockSpec((_BN, 128), lambda b: (b, 0)),
            pl.BlockSpec((_BN, 1), lambda b: (b, 0)),
            pl.BlockSpec((1, 128), lambda b: (0, 0)),
        ],
        out_specs=pl.BlockSpec((_BN, 128), lambda b: (b, 0)),
        out_shape=jax.ShapeDtypeStruct((NPAD, 128), jnp.float32),
    )(a2, indeg, b2row)


def kernel(blob, edge_index, W_ih, W_hh, b_ih, b_hh, W0, b0, W1, b1, W2, b2):
    src = edge_index[0].astype(jnp.int32)
    dst = edge_index[1].astype(jnp.int32)
    order_d = jnp.argsort(dst)
    dst_s = dst[order_d]
    src_s = src[order_d]
    src_ss = jnp.sort(src)
    node_bnds = jnp.arange(0, NPAD + 1, NB, dtype=jnp.int32)
    bounds_d = jnp.pad(
        jnp.searchsorted(dst_s, node_bnds).astype(jnp.int32), (0, 7))
    bounds_s = jnp.pad(
        jnp.searchsorted(src_ss, node_bnds).astype(jnp.int32), (0, 7))

    deg = _sc_degrees(dst_s, src_ss, bounds_d, bounds_s).reshape(2, NPAD)
    indeg = deg[0].reshape(NPAD, 1)
    outdeg = deg[1].reshape(NPAD, 1)

    blob2 = jnp.pad(blob[:, :, 0], ((0, NPAD - N), (0, 0)))
    wih = W_ih[:, 0][None, :]
    whhT = W_hh.T
    bsum = (b_ih + b_hh)[None, :]
    z0 = _lstm(blob2, wih, whhT, bsum, outdeg)

    a0 = _sc_agg256(z0.reshape(T * NPAD, H), src_s, dst_s,
                    bounds_d).reshape(T, NPAD, H)
    z1 = _conv(a0, indeg, outdeg, W0, b0[None, :])
    a1 = _sc_agg256(z1.reshape(T * NPAD, H), src_s, dst_s,
                    bounds_d).reshape(T, NPAD, H)

    w2s = jnp.zeros((T, H, 128), jnp.float32).at[
        jnp.arange(T), :, jnp.arange(T)].set(
        jnp.broadcast_to(W2[:, 0], (T, H)))
    z2 = _conv2(a1, indeg, outdeg, W1, b1[None, :], w2s)

    a2 = _sc_agg16(z2, src_s, dst_s, bounds_d).reshape(NPAD, 128)
    b2row = jnp.broadcast_to(b2, (128,))[None, :]
    out = _final(a2, indeg, b2row)
    return out[:N, :T]


# speedup vs baseline: 1.9108x; 1.8787x over previous
import functools

import jax
import jax.numpy as jnp
from jax import lax
from jax.experimental import pallas as pl
from jax.experimental.pallas import tpu as pltpu
from jax.experimental.pallas import tpu_sc as plsc

N = 10000
E = 160000
H = 256
T = 12
NPAD = 10240
NW = 32
NB = NPAD // NW
NC = 2

_mesh = plsc.VectorSubcoreMesh(core_axis_name="c", subcore_axis_name="s")


def _scalar_at(ref, i):
    v = plsc.load_gather(ref, [jnp.full((16,), i, jnp.int32)])
    return jnp.max(v)


@functools.partial(
    pl.kernel,
    out_type=jax.ShapeDtypeStruct((2 * NPAD,), jnp.float32),
    mesh=_mesh,
    compiler_params=pltpu.CompilerParams(needs_layout_passes=False),
    scratch_types=[
        pltpu.VMEM((16 * (NB + 1),), jnp.float32),
        pltpu.VMEM((NB,), jnp.float32),
        pltpu.VMEM((40,), jnp.int32),
        pltpu.VMEM((16,), jnp.int32),
    ],
)
def _sc_degrees(dst_s, src_ss, bounds_d, bounds_s, out, hist, degbuf, bvm, ivm):
    wid = lax.axis_index("s") * NC + lax.axis_index("c")
    iota = lax.iota(jnp.int32, 16)
    ones = jnp.full((16,), 1.0, jnp.float32)
    zeros16 = jnp.zeros((16,), jnp.float32)
    for p, (arr, bnds) in enumerate(((dst_s, bounds_d), (src_ss, bounds_s))):
        pltpu.sync_copy(bnds, bvm)
        s0 = _scalar_at(bvm, wid)
        s1 = _scalar_at(bvm, wid + 1)

        def zbody(i, _):
            hist[pl.ds(i * 16, 16)] = zeros16
            return 0
        lax.fori_loop(0, NB + 1, zbody, 0)

        base_node = wid * NB

        def cbody(c):
            cc = pl.multiple_of(c, 16)
            pltpu.sync_copy(arr.at[pl.ds(cc, 16)], ivm)
            d = ivm[...]
            pos = c + iota
            valid = (pos >= s0) & (pos < s1)
            local = jnp.where(valid, d - base_node, NB)
            idx = iota * (NB + 1) + local
            plsc.addupdate_scatter(hist, [idx], ones)
            return c + 16

        c0 = (s0 // 16) * 16
        lax.while_loop(lambda c: c < s1, cbody, c0)

        for g in range(NB // 16):
            acc = zeros16
            for lane in range(16):
                acc = acc + hist[pl.ds(lane * (NB + 1) + g * 16, 16)]
            degbuf[pl.ds(g * 16, 16)] = acc
        pltpu.sync_copy(
            degbuf,
            out.at[pl.ds(pl.multiple_of(p * NPAD + wid * NB, NB), NB)])


SE = 256
GB = 64


@functools.partial(
    pl.kernel,
    out_type=jax.ShapeDtypeStruct((T * NPAD * H,), jnp.float32),
    mesh=_mesh,
    compiler_params=pltpu.CompilerParams(needs_layout_passes=False),
    scratch_types=[
        pltpu.VMEM(((NB + 1) * H,), jnp.float32),
        pltpu.VMEM((2 * GB, H), jnp.float32),
        pltpu.VMEM((40,), jnp.int32),
        pltpu.VMEM((SE,), jnp.int32),
        pltpu.VMEM((SE,), jnp.int32),
        pltpu.VMEM((SE,), jnp.int32),
        pltpu.SemaphoreType.DMA,
        pltpu.SemaphoreType.DMA,
    ],
)
def _sc_agg256(table, src_s, dst_s, bounds, out,
               acc, stage, bvm, svm, dvm, gidx, sem0, sem1):
    wid = lax.axis_index("s") * NC + lax.axis_index("c")
    iota = lax.iota(jnp.int32, 16)
    zeros16 = jnp.zeros((16,), jnp.float32)
    sems = (sem0, sem1)
    pltpu.sync_copy(bounds, bvm)
    s0 = _scalar_at(bvm, wid)
    s1 = _scalar_at(bvm, wid + 1)
    base_node = wid * NB
    c0 = (s0 // SE) * SE

    def tbody(t, _):
        def zbody(i, _):
            acc[pl.ds(i * 16, 16)] = zeros16
            return 0
        lax.fori_loop(0, (NB + 1) * H // 16, zbody, 0)

        def cbody(c):
            cc = pl.multiple_of(c, SE)
            pltpu.sync_copy(src_s.at[pl.ds(cc, SE)], svm)
            pltpu.sync_copy(dst_s.at[pl.ds(cc, SE)], dvm)

            def ibody(q, _):
                qq = pl.multiple_of(q * 16, 16)
                gidx[pl.ds(qq, 16)] = svm[pl.ds(qq, 16)] + (t * NPAD)
                return 0
            lax.fori_loop(0, SE // 16, ibody, 0)

            def issue(m):
                return pltpu.async_copy(
                    table.at[gidx.at[pl.ds(m * GB, GB)]],
                    stage.at[pl.ds((m % 2) * GB, GB)], sems[m % 2])

            def accum(m):
                def gbody(g, _):
                    off = pl.multiple_of(m * GB + g * 16, 16)
                    dvec = dvm[pl.ds(off, 16)]
                    pos = c + off + iota
                    valid = (pos >= s0) & (pos < s1)
                    local = jnp.where(valid, dvec - base_node, NB)
                    erow = (m % 2) * GB + g * 16
                    for j in range(16):
                        lj = jnp.max(jnp.where(iota == j, local, 0))
                        rowbase = jnp.full((16,), lj * H, jnp.int32)
                        for k in range(H // 16):
                            idxv = rowbase + (iota + k * 16)
                            plsc.addupdate_scatter(
                                acc, [idxv],
                                stage[erow + j, pl.ds(k * 16, 16)])
                    return 0
                lax.fori_loop(0, GB // 16, gbody, 0)

            d = issue(0)
            descs = {0: d}
            for m in range(SE // GB):
                if m + 1 < SE // GB:
                    descs[m + 1] = issue(m + 1)
                descs[m].wait()
                accum(m)
            return c + SE

        lax.while_loop(lambda c: c < s1, cbody, c0)
        pltpu.sync_copy(
            acc.at[pl.ds(0, NB * H)],
            out.at[pl.ds(pl.multiple_of((t * NPAD + wid * NB) * H, NB * H),
                         NB * H)])
        return 0

    lax.fori_loop(0, T, tbody, 0)


@functools.partial(
    pl.kernel,
    out_type=jax.ShapeDtypeStruct((NPAD * 128,), jnp.float32),
    mesh=_mesh,
    compiler_params=pltpu.CompilerParams(needs_layout_passes=False),
    scratch_types=[
        pltpu.VMEM(((NB + 1) * 128,), jnp.float32),
        pltpu.VMEM((2 * GB, 128), jnp.float32),
        pltpu.VMEM((40,), jnp.int32),
        pltpu.VMEM((SE,), jnp.int32),
        pltpu.VMEM((SE,), jnp.int32),
        pltpu.SemaphoreType.DMA,
        pltpu.SemaphoreType.DMA,
    ],
)
def _sc_agg16(table, src_s, dst_s, bounds, out,
              acc, stage, bvm, svm, dvm, sem0, sem1):
    wid = lax.axis_index("s") * NC + lax.axis_index("c")
    iota = lax.iota(jnp.int32, 16)
    zeros16 = jnp.zeros((16,), jnp.float32)
    sems = (sem0, sem1)
    pltpu.sync_copy(bounds, bvm)
    s0 = _scalar_at(bvm, wid)
    s1 = _scalar_at(bvm, wid + 1)
    base_node = wid * NB

    def zbody(i, _):
        acc[pl.ds(i * 16, 16)] = zeros16
        return 0
    lax.fori_loop(0, (NB + 1) * 128 // 16, zbody, 0)

    def cbody(c):
        cc = pl.multiple_of(c, SE)
        pltpu.sync_copy(src_s.at[pl.ds(cc, SE)], svm)
        pltpu.sync_copy(dst_s.at[pl.ds(cc, SE)], dvm)

        def issue(m):
            return pltpu.async_copy(
                table.at[svm.at[pl.ds(m * GB, GB)]],
                stage.at[pl.ds((m % 2) * GB, GB)], sems[m % 2])

        def accum(m):
            def gbody(g, _):
                off = pl.multiple_of(m * GB + g * 16, 16)
                dvec = dvm[pl.ds(off, 16)]
                pos = c + off + iota
                valid = (pos >= s0) & (pos < s1)
                local = jnp.where(valid, dvec - base_node, NB)
                erow = (m % 2) * GB + g * 16
                for j in range(16):
                    lj = jnp.max(jnp.where(iota == j, local, 0))
                    rowbase = jnp.full((16,), lj * 128, jnp.int32)
                    for k in range(8):
                        idxv = rowbase + (iota + k * 16)
                        plsc.addupdate_scatter(
                            acc, [idxv],
                            stage[erow + j, pl.ds(k * 16, 16)])
                return 0
            lax.fori_loop(0, GB // 16, gbody, 0)

        d = issue(0)
        descs = {0: d}
        for m in range(SE // GB):
            if m + 1 < SE // GB:
                descs[m + 1] = issue(m + 1)
            descs[m].wait()
            accum(m)
        return c + SE

    c0 = (s0 // SE) * SE
    lax.while_loop(lambda c: c < s1, cbody, c0)
    pltpu.sync_copy(acc.at[pl.ds(0, NB * 128)],
                    out.at[pl.ds(pl.multiple_of(wid * NB * 128, NB * 128),
                                 NB * 128)])


_BN = 512
_NBLK = NPAD // _BN


def _lstm_body(x_ref, wih_ref, whh_ref, b_ref, deg_ref, z_ref):
    bn = x_ref.shape[0]
    h = jnp.zeros((bn, H), jnp.float32)
    c = jnp.zeros((bn, H), jnp.float32)
    nsrc = lax.rsqrt(jnp.maximum(deg_ref[...], 1.0))
    x = x_ref[...]
    wih = wih_ref[...]
    b = b_ref[...]
    whh = whh_ref[...]
    for t in range(T):
        gates = (x[:, t:t + 1] * wih
                 + jnp.dot(h, whh, preferred_element_type=jnp.float32,
                 precision=lax.Precision.HIGHEST) + b)
        i_g = jax.nn.sigmoid(gates[:, 0 * H:1 * H])
        f_g = jax.nn.sigmoid(gates[:, 1 * H:2 * H])
        g_g = jnp.tanh(gates[:, 2 * H:3 * H])
        o_g = jax.nn.sigmoid(gates[:, 3 * H:4 * H])
        c = f_g * c + i_g * g_g
        h = o_g * jnp.tanh(c)
        z_ref[t] = nsrc * h


def _lstm(blob2, wih, whhT, bsum, outdeg):
    return pl.pallas_call(
        _lstm_body,
        grid=(_NBLK,),
        in_specs=[
            pl.BlockSpec((_BN, T), lambda b: (b, 0)),
            pl.BlockSpec((1, 4 * H), lambda b: (0, 0)),
            pl.BlockSpec((H, 4 * H), lambda b: (0, 0)),
            pl.BlockSpec((1, 4 * H), lambda b: (0, 0)),
            pl.BlockSpec((_BN, 1), lambda b: (b, 0)),
        ],
        out_specs=pl.BlockSpec((T, _BN, H), lambda b: (0, b, 0)),
        out_shape=jax.ShapeDtypeStruct((T, NPAD, H), jnp.float32),
    )(blob2, wih, whhT, bsum, outdeg)


def _conv_body(a_ref, ideg_ref, odeg_ref, w_ref, b_ref, z_ref):
    ndst = lax.rsqrt(jnp.maximum(ideg_ref[...], 1.0))
    nsrc = lax.rsqrt(jnp.maximum(odeg_ref[...], 1.0))
    a = a_ref[0]
    hh = jax.nn.relu(ndst * jnp.dot(a, w_ref[...],
                                    preferred_element_type=jnp.float32,
                 precision=lax.Precision.HIGHEST)
                     + b_ref[...])
    z_ref[0] = nsrc * hh


def _conv(a3, indeg, outdeg, w, brow):
    return pl.pallas_call(
        _conv_body,
        grid=(T, _NBLK),
        in_specs=[
            pl.BlockSpec((1, _BN, H), lambda t, b: (t, b, 0)),
            pl.BlockSpec((_BN, 1), lambda t, b: (b, 0)),
            pl.BlockSpec((_BN, 1), lambda t, b: (b, 0)),
            pl.BlockSpec((H, H), lambda t, b: (0, 0)),
            pl.BlockSpec((1, H), lambda t, b: (0, 0)),
        ],
        out_specs=pl.BlockSpec((1, _BN, H), lambda t, b: (t, b, 0)),
        out_shape=jax.ShapeDtypeStruct((T, NPAD, H), jnp.float32),
    )(a3, indeg, outdeg, w, brow)


def _conv2_body(a_ref, ideg_ref, odeg_ref, w1_ref, b1_ref, w2s_ref, z_ref):
    ndst = lax.rsqrt(jnp.maximum(ideg_ref[...], 1.0))
    nsrc = lax.rsqrt(jnp.maximum(odeg_ref[...], 1.0))
    acc = jnp.zeros((a_ref.shape[1], 128), jnp.float32)
    for t in range(T):
        h2 = jax.nn.relu(ndst * jnp.dot(a_ref[t], w1_ref[...],
                                        preferred_element_type=jnp.float32,
                 precision=lax.Precision.HIGHEST)
                         + b1_ref[...])
        acc = acc + jnp.dot(h2, w2s_ref[t], preferred_element_type=jnp.float32,
                 precision=lax.Precision.HIGHEST)
    z_ref[...] = nsrc * acc


def _conv2(a3, indeg, outdeg, w1, b1row, w2s):
    return pl.pallas_call(
        _conv2_body,
        grid=(_NBLK,),
        in_specs=[
            pl.BlockSpec((T, _BN, H), lambda b: (0, b, 0)),
            pl.BlockSpec((_BN, 1), lambda b: (b, 0)),
            pl.BlockSpec((_BN, 1), lambda b: (b, 0)),
            pl.BlockSpec((H, H), lambda b: (0, 0)),
            pl.BlockSpec((1, H), lambda b: (0, 0)),
            pl.BlockSpec((T, H, 128), lambda b: (0, 0, 0)),
        ],
        out_specs=pl.BlockSpec((_BN, 128), lambda b: (b, 0)),
        out_shape=jax.ShapeDtypeStruct((NPAD, 128), jnp.float32),
    )(a3, indeg, outdeg, w1, b1row, w2s)


def _final_body(a_ref, ideg_ref, b2_ref, o_ref):
    ndst = lax.rsqrt(jnp.maximum(ideg_ref[...], 1.0))
    o_ref[...] = ndst * a_ref[...] + b2_ref[...]


def _final(a2, indeg, b2row):
    return pl.pallas_call(
        _final_body,
        grid=(_NBLK,),
        in_specs=[
            pl.BlockSpec((_BN, 128), lambda b: (b, 0)),
            pl.BlockSpec((_BN, 1), lambda b: (b, 0)),
            pl.BlockSpec((1, 128), lambda b: (0, 0)),
        ],
        out_specs=pl.BlockSpec((_BN, 128), lambda b: (b, 0)),
        out_shape=jax.ShapeDtypeStruct((NPAD, 128), jnp.float32),
    )(a2, indeg, b2row)


def kernel(blob, edge_index, W_ih, W_hh, b_ih, b_hh, W0, b0, W1, b1, W2, b2):
    src = edge_index[0].astype(jnp.int32)
    dst = edge_index[1].astype(jnp.int32)
    order_d = jnp.argsort(dst)
    dst_s = dst[order_d]
    src_s = src[order_d]
    src_ss = jnp.sort(src)
    node_bnds = jnp.arange(0, NPAD + 1, NB, dtype=jnp.int32)
    bounds_d = jnp.pad(
        jnp.searchsorted(dst_s, node_bnds).astype(jnp.int32), (0, 7))
    bounds_s = jnp.pad(
        jnp.searchsorted(src_ss, node_bnds).astype(jnp.int32), (0, 7))

    deg = _sc_degrees(dst_s, src_ss, bounds_d, bounds_s).reshape(2, NPAD)
    indeg = deg[0].reshape(NPAD, 1)
    outdeg = deg[1].reshape(NPAD, 1)

    blob2 = jnp.pad(blob[:, :, 0], ((0, NPAD - N), (0, 0)))
    wih = W_ih[:, 0][None, :]
    whhT = W_hh.T
    bsum = (b_ih + b_hh)[None, :]
    z0 = _lstm(blob2, wih, whhT, bsum, outdeg)

    a0 = _sc_agg256(z0.reshape(T * NPAD, H), src_s, dst_s,
                    bounds_d).reshape(T, NPAD, H)
    z1 = _conv(a0, indeg, outdeg, W0, b0[None, :])
    a1 = _sc_agg256(z1.reshape(T * NPAD, H), src_s, dst_s,
                    bounds_d).reshape(T, NPAD, H)

    w2s = jnp.zeros((T, H, 128), jnp.float32).at[
        jnp.arange(T), :, jnp.arange(T)].set(
        jnp.broadcast_to(W2[:, 0], (T, H)))
    z2 = _conv2(a1, indeg, outdeg, W1, b1[None, :], w2s)

    a2 = _sc_agg16(z2, src_s, dst_s, bounds_d).reshape(NPAD, 128)
    b2row = jnp.broadcast_to(b2, (128,))[None, :]
    out = _final(a2, indeg, b2row)
    return out[:N, :T]


# speedup vs baseline: 3.1566x; 1.6520x over previous
import functools

import jax
import jax.numpy as jnp
from jax import lax
from jax.experimental import pallas as pl
from jax.experimental.pallas import tpu as pltpu
from jax.experimental.pallas import tpu_sc as plsc

N = 10000
E = 160000
H = 256
T = 12
NPAD = 10240
NW = 32
NB = NPAD // NW
NC = 2

_mesh = plsc.VectorSubcoreMesh(core_axis_name="c", subcore_axis_name="s")


def _vbroadcast(x, j):
    idx = jnp.full((16, 1), j, jnp.int32)
    dn = lax.GatherDimensionNumbers(offset_dims=(), collapsed_slice_dims=(0,),
                                    start_index_map=(0,))
    return lax.gather(x, idx, dn, slice_sizes=(1,),
                      mode=lax.GatherScatterMode.PROMISE_IN_BOUNDS)


def _scalar_at(ref, i):
    v = plsc.load_gather(ref, [jnp.full((16,), i, jnp.int32)])
    return jnp.max(v)


@functools.partial(
    pl.kernel,
    out_type=jax.ShapeDtypeStruct((2 * NPAD,), jnp.float32),
    mesh=_mesh,
    compiler_params=pltpu.CompilerParams(needs_layout_passes=False),
    scratch_types=[
        pltpu.VMEM((16 * (NB + 1),), jnp.float32),
        pltpu.VMEM((NB,), jnp.float32),
        pltpu.VMEM((40,), jnp.int32),
        pltpu.VMEM((16,), jnp.int32),
    ],
)
def _sc_degrees(dst_s, src_ss, bounds_d, bounds_s, out, hist, degbuf, bvm, ivm):
    wid = lax.axis_index("s") * NC + lax.axis_index("c")
    iota = lax.iota(jnp.int32, 16)
    ones = jnp.full((16,), 1.0, jnp.float32)
    zeros16 = jnp.zeros((16,), jnp.float32)
    for p, (arr, bnds) in enumerate(((dst_s, bounds_d), (src_ss, bounds_s))):
        pltpu.sync_copy(bnds, bvm)
        s0 = _scalar_at(bvm, wid)
        s1 = _scalar_at(bvm, wid + 1)

        def zbody(i, _):
            hist[pl.ds(i * 16, 16)] = zeros16
            return 0
        lax.fori_loop(0, NB + 1, zbody, 0)

        base_node = wid * NB

        def cbody(c):
            cc = pl.multiple_of(c, 16)
            pltpu.sync_copy(arr.at[pl.ds(cc, 16)], ivm)
            d = ivm[...]
            pos = c + iota
            valid = (pos >= s0) & (pos < s1)
            local = jnp.where(valid, d - base_node, NB)
            idx = iota * (NB + 1) + local
            plsc.addupdate_scatter(hist, [idx], ones)
            return c + 16

        c0 = (s0 // 16) * 16
        lax.while_loop(lambda c: c < s1, cbody, c0)

        for g in range(NB // 16):
            acc = zeros16
            for lane in range(16):
                acc = acc + hist[pl.ds(lane * (NB + 1) + g * 16, 16)]
            degbuf[pl.ds(g * 16, 16)] = acc
        pltpu.sync_copy(
            degbuf,
            out.at[pl.ds(pl.multiple_of(p * NPAD + wid * NB, NB), NB)])


SE = 256
GB = 64


@functools.partial(
    pl.kernel,
    out_type=jax.ShapeDtypeStruct((T * NPAD * H,), jnp.float32),
    mesh=_mesh,
    compiler_params=pltpu.CompilerParams(needs_layout_passes=False),
    scratch_types=[
        pltpu.VMEM(((NB + 1) * H,), jnp.float32),
        pltpu.VMEM((2 * GB, H), jnp.float32),
        pltpu.VMEM((40,), jnp.int32),
        pltpu.VMEM((SE,), jnp.int32),
        pltpu.VMEM((SE,), jnp.int32),
        pltpu.VMEM((SE,), jnp.int32),
        pltpu.SemaphoreType.DMA,
        pltpu.SemaphoreType.DMA,
    ],
)
def _sc_agg256(table, src_s, dst_s, bounds, out,
               acc, stage, bvm, svm, dvm, gidx, sem0, sem1):
    wid = lax.axis_index("s") * NC + lax.axis_index("c")
    iota = lax.iota(jnp.int32, 16)
    zeros16 = jnp.zeros((16,), jnp.float32)
    sems = (sem0, sem1)
    pltpu.sync_copy(bounds, bvm)
    s0 = _scalar_at(bvm, wid)
    s1 = _scalar_at(bvm, wid + 1)
    base_node = wid * NB
    c0 = (s0 // SE) * SE

    def tbody(t, _):
        def zbody(i, _):
            acc[pl.ds(i * 16, 16)] = zeros16
            return 0
        lax.fori_loop(0, (NB + 1) * H // 16, zbody, 0)

        def cbody(c):
            cc = pl.multiple_of(c, SE)
            pltpu.sync_copy(src_s.at[pl.ds(cc, SE)], svm)
            pltpu.sync_copy(dst_s.at[pl.ds(cc, SE)], dvm)

            def ibody(q, _):
                qq = pl.multiple_of(q * 16, 16)
                gidx[pl.ds(qq, 16)] = svm[pl.ds(qq, 16)] + (t * NPAD)
                return 0
            lax.fori_loop(0, SE // 16, ibody, 0)

            def issue(m):
                return pltpu.async_copy(
                    table.at[gidx.at[pl.ds(m * GB, GB)]],
                    stage.at[pl.ds((m % 2) * GB, GB)], sems[m % 2])

            def accum(m):
                def gbody(g, _):
                    off = pl.multiple_of(m * GB + g * 16, 16)
                    dvec = dvm[pl.ds(off, 16)]
                    pos = c + off + iota
                    valid = (pos >= s0) & (pos < s1)
                    local = jnp.where(valid, dvec - base_node, NB)
                    erow = (m % 2) * GB + g * 16
                    lrow = local * H
                    for j in range(16):
                        rowbase = _vbroadcast(lrow, j)
                        vals = [stage[erow + j, pl.ds(k * 16, 16)]
                                for k in range(H // 16)]
                        for k in range(H // 16):
                            idxv = rowbase + (iota + k * 16)
                            plsc.addupdate_scatter(acc, [idxv], vals[k])
                    return 0
                lax.fori_loop(0, GB // 16, gbody, 0)

            d = issue(0)
            descs = {0: d}
            for m in range(SE // GB):
                if m + 1 < SE // GB:
                    descs[m + 1] = issue(m + 1)
                descs[m].wait()
                accum(m)
            return c + SE

        lax.while_loop(lambda c: c < s1, cbody, c0)
        pltpu.sync_copy(
            acc.at[pl.ds(0, NB * H)],
            out.at[pl.ds(pl.multiple_of((t * NPAD + wid * NB) * H, NB * H),
                         NB * H)])
        return 0

    lax.fori_loop(0, T, tbody, 0)


@functools.partial(
    pl.kernel,
    out_type=jax.ShapeDtypeStruct((NPAD * 128,), jnp.float32),
    mesh=_mesh,
    compiler_params=pltpu.CompilerParams(needs_layout_passes=False),
    scratch_types=[
        pltpu.VMEM(((NB + 1) * 128,), jnp.float32),
        pltpu.VMEM((2 * GB, 128), jnp.float32),
        pltpu.VMEM((40,), jnp.int32),
        pltpu.VMEM((SE,), jnp.int32),
        pltpu.VMEM((SE,), jnp.int32),
        pltpu.SemaphoreType.DMA,
        pltpu.SemaphoreType.DMA,
    ],
)
def _sc_agg16(table, src_s, dst_s, bounds, out,
              acc, stage, bvm, svm, dvm, sem0, sem1):
    wid = lax.axis_index("s") * NC + lax.axis_index("c")
    iota = lax.iota(jnp.int32, 16)
    zeros16 = jnp.zeros((16,), jnp.float32)
    sems = (sem0, sem1)
    pltpu.sync_copy(bounds, bvm)
    s0 = _scalar_at(bvm, wid)
    s1 = _scalar_at(bvm, wid + 1)
    base_node = wid * NB

    def zbody(i, _):
        acc[pl.ds(i * 16, 16)] = zeros16
        return 0
    lax.fori_loop(0, (NB + 1) * 128 // 16, zbody, 0)

    def cbody(c):
        cc = pl.multiple_of(c, SE)
        pltpu.sync_copy(src_s.at[pl.ds(cc, SE)], svm)
        pltpu.sync_copy(dst_s.at[pl.ds(cc, SE)], dvm)

        def issue(m):
            return pltpu.async_copy(
                table.at[svm.at[pl.ds(m * GB, GB)]],
                stage.at[pl.ds((m % 2) * GB, GB)], sems[m % 2])

        def accum(m):
            def gbody(g, _):
                off = pl.multiple_of(m * GB + g * 16, 16)
                dvec = dvm[pl.ds(off, 16)]
                pos = c + off + iota
                valid = (pos >= s0) & (pos < s1)
                local = jnp.where(valid, dvec - base_node, NB)
                erow = (m % 2) * GB + g * 16
                lrow = local * 128
                for j in range(16):
                    rowbase = _vbroadcast(lrow, j)
                    vals = [stage[erow + j, pl.ds(k * 16, 16)]
                            for k in range(8)]
                    for k in range(8):
                        idxv = rowbase + (iota + k * 16)
                        plsc.addupdate_scatter(acc, [idxv], vals[k])
                return 0
            lax.fori_loop(0, GB // 16, gbody, 0)

        d = issue(0)
        descs = {0: d}
        for m in range(SE // GB):
            if m + 1 < SE // GB:
                descs[m + 1] = issue(m + 1)
            descs[m].wait()
            accum(m)
        return c + SE

    c0 = (s0 // SE) * SE
    lax.while_loop(lambda c: c < s1, cbody, c0)
    pltpu.sync_copy(acc.at[pl.ds(0, NB * 128)],
                    out.at[pl.ds(pl.multiple_of(wid * NB * 128, NB * 128),
                                 NB * 128)])


_BN = 512
_NBLK = NPAD // _BN


def _lstm_body(x_ref, wih_ref, whh_ref, b_ref, deg_ref, z_ref):
    bn = x_ref.shape[0]
    h = jnp.zeros((bn, H), jnp.float32)
    c = jnp.zeros((bn, H), jnp.float32)
    nsrc = lax.rsqrt(jnp.maximum(deg_ref[...], 1.0))
    x = x_ref[...]
    wih = wih_ref[...]
    b = b_ref[...]
    whh = whh_ref[...]
    for t in range(T):
        gates = (x[:, t:t + 1] * wih
                 + jnp.dot(h, whh, preferred_element_type=jnp.float32,
                 precision=lax.Precision.HIGHEST) + b)
        i_g = jax.nn.sigmoid(gates[:, 0 * H:1 * H])
        f_g = jax.nn.sigmoid(gates[:, 1 * H:2 * H])
        g_g = jnp.tanh(gates[:, 2 * H:3 * H])
        o_g = jax.nn.sigmoid(gates[:, 3 * H:4 * H])
        c = f_g * c + i_g * g_g
        h = o_g * jnp.tanh(c)
        z_ref[t] = nsrc * h


def _lstm(blob2, wih, whhT, bsum, outdeg):
    return pl.pallas_call(
        _lstm_body,
        grid=(_NBLK,),
        in_specs=[
            pl.BlockSpec((_BN, T), lambda b: (b, 0)),
            pl.BlockSpec((1, 4 * H), lambda b: (0, 0)),
            pl.BlockSpec((H, 4 * H), lambda b: (0, 0)),
            pl.BlockSpec((1, 4 * H), lambda b: (0, 0)),
            pl.BlockSpec((_BN, 1), lambda b: (b, 0)),
        ],
        out_specs=pl.BlockSpec((T, _BN, H), lambda b: (0, b, 0)),
        out_shape=jax.ShapeDtypeStruct((T, NPAD, H), jnp.float32),
    )(blob2, wih, whhT, bsum, outdeg)


def _conv_body(a_ref, ideg_ref, odeg_ref, w_ref, b_ref, z_ref):
    ndst = lax.rsqrt(jnp.maximum(ideg_ref[...], 1.0))
    nsrc = lax.rsqrt(jnp.maximum(odeg_ref[...], 1.0))
    a = a_ref[0]
    hh = jax.nn.relu(ndst * jnp.dot(a, w_ref[...],
                                    preferred_element_type=jnp.float32,
                 precision=lax.Precision.HIGHEST)
                     + b_ref[...])
    z_ref[0] = nsrc * hh


def _conv(a3, indeg, outdeg, w, brow):
    return pl.pallas_call(
        _conv_body,
        grid=(T, _NBLK),
        in_specs=[
            pl.BlockSpec((1, _BN, H), lambda t, b: (t, b, 0)),
            pl.BlockSpec((_BN, 1), lambda t, b: (b, 0)),
            pl.BlockSpec((_BN, 1), lambda t, b: (b, 0)),
            pl.BlockSpec((H, H), lambda t, b: (0, 0)),
            pl.BlockSpec((1, H), lambda t, b: (0, 0)),
        ],
        out_specs=pl.BlockSpec((1, _BN, H), lambda t, b: (t, b, 0)),
        out_shape=jax.ShapeDtypeStruct((T, NPAD, H), jnp.float32),
    )(a3, indeg, outdeg, w, brow)


def _conv2_body(a_ref, ideg_ref, odeg_ref, w1_ref, b1_ref, w2s_ref, z_ref):
    ndst = lax.rsqrt(jnp.maximum(ideg_ref[...], 1.0))
    nsrc = lax.rsqrt(jnp.maximum(odeg_ref[...], 1.0))
    acc = jnp.zeros((a_ref.shape[1], 128), jnp.float32)
    for t in range(T):
        h2 = jax.nn.relu(ndst * jnp.dot(a_ref[t], w1_ref[...],
                                        preferred_element_type=jnp.float32,
                 precision=lax.Precision.HIGHEST)
                         + b1_ref[...])
        acc = acc + jnp.dot(h2, w2s_ref[t], preferred_element_type=jnp.float32,
                 precision=lax.Precision.HIGHEST)
    z_ref[...] = nsrc * acc


def _conv2(a3, indeg, outdeg, w1, b1row, w2s):
    return pl.pallas_call(
        _conv2_body,
        grid=(_NBLK,),
        in_specs=[
            pl.BlockSpec((T, _BN, H), lambda b: (0, b, 0)),
            pl.BlockSpec((_BN, 1), lambda b: (b, 0)),
            pl.BlockSpec((_BN, 1), lambda b: (b, 0)),
            pl.BlockSpec((H, H), lambda b: (0, 0)),
            pl.BlockSpec((1, H), lambda b: (0, 0)),
            pl.BlockSpec((T, H, 128), lambda b: (0, 0, 0)),
        ],
        out_specs=pl.BlockSpec((_BN, 128), lambda b: (b, 0)),
        out_shape=jax.ShapeDtypeStruct((NPAD, 128), jnp.float32),
    )(a3, indeg, outdeg, w1, b1row, w2s)


def _final_body(a_ref, ideg_ref, b2_ref, o_ref):
    ndst = lax.rsqrt(jnp.maximum(ideg_ref[...], 1.0))
    o_ref[...] = ndst * a_ref[...] + b2_ref[...]


def _final(a2, indeg, b2row):
    return pl.pallas_call(
        _final_body,
        grid=(_NBLK,),
        in_specs=[
            pl.BlockSpec((_BN, 128), lambda b: (b, 0)),
            pl.BlockSpec((_BN, 1), lambda b: (b, 0)),
            pl.BlockSpec((1, 128), lambda b: (0, 0)),
        ],
        out_specs=pl.BlockSpec((_BN, 128), lambda b: (b, 0)),
        out_shape=jax.ShapeDtypeStruct((NPAD, 128), jnp.float32),
    )(a2, indeg, b2row)


def kernel(blob, edge_index, W_ih, W_hh, b_ih, b_hh, W0, b0, W1, b1, W2, b2):
    src = edge_index[0].astype(jnp.int32)
    dst = edge_index[1].astype(jnp.int32)
    order_d = jnp.argsort(dst)
    dst_s = dst[order_d]
    src_s = src[order_d]
    src_ss = jnp.sort(src)
    node_bnds = jnp.arange(0, NPAD + 1, NB, dtype=jnp.int32)
    bounds_d = jnp.pad(
        jnp.searchsorted(dst_s, node_bnds).astype(jnp.int32), (0, 7))
    bounds_s = jnp.pad(
        jnp.searchsorted(src_ss, node_bnds).astype(jnp.int32), (0, 7))

    deg = _sc_degrees(dst_s, src_ss, bounds_d, bounds_s).reshape(2, NPAD)
    indeg = deg[0].reshape(NPAD, 1)
    outdeg = deg[1].reshape(NPAD, 1)

    blob2 = jnp.pad(blob[:, :, 0], ((0, NPAD - N), (0, 0)))
    wih = W_ih[:, 0][None, :]
    whhT = W_hh.T
    bsum = (b_ih + b_hh)[None, :]
    z0 = _lstm(blob2, wih, whhT, bsum, outdeg)

    a0 = _sc_agg256(z0.reshape(T * NPAD, H), src_s, dst_s,
                    bounds_d).reshape(T, NPAD, H)
    z1 = _conv(a0, indeg, outdeg, W0, b0[None, :])
    a1 = _sc_agg256(z1.reshape(T * NPAD, H), src_s, dst_s,
                    bounds_d).reshape(T, NPAD, H)

    w2s = jnp.zeros((T, H, 128), jnp.float32).at[
        jnp.arange(T), :, jnp.arange(T)].set(
        jnp.broadcast_to(W2[:, 0], (T, H)))
    z2 = _conv2(a1, indeg, outdeg, W1, b1[None, :], w2s)

    a2 = _sc_agg16(z2, src_s, dst_s, bounds_d).reshape(NPAD, 128)
    b2row = jnp.broadcast_to(b2, (128,))[None, :]
    out = _final(a2, indeg, b2row)
    return out[:N, :T]
